# Initial kernel scaffold; baseline (speedup 1.0000x reference)
import jax
import jax.numpy as jnp
from jax.experimental import pallas as pl


def kernel(inputs, emb, W, b):
    raise NotImplementedError("write your pallas kernel here")


# speedup vs baseline: 3.2985x; 3.2985x over previous
import functools

import jax
import jax.numpy as jnp
from jax import lax
from jax.experimental import pallas as pl
from jax.experimental.pallas import tpu as pltpu
from jax.experimental.pallas import tpu_sc as plsc

EMB = 16
NC = 2
NS = 16
NW = NC * NS
BL = 128
KJ = 16
B = KJ * BL


@functools.lru_cache(maxsize=None)
def _make_sc_sum(n_rows: int):
    chunk_rows = n_rows // NW
    assert chunk_rows * NW == n_rows
    R = chunk_rows // KJ
    assert R * KJ == chunk_rows and R % 2 == 0
    NK = R // 2

    mesh = plsc.VectorSubcoreMesh(
        core_axis_name="c", subcore_axis_name="s",
        num_cores=NC, num_subcores=NS)

    @functools.partial(
        pl.kernel,
        out_type=jax.ShapeDtypeStruct((NW, EMB), jnp.float32),
        mesh=mesh,
        compiler_params=pltpu.CompilerParams(use_tc_tiling_on_sc=False),
        scratch_types=[
            pltpu.VMEM((KJ, BL), jnp.int32),
            pltpu.VMEM((KJ, BL), jnp.int32),
            pltpu.VMEM((B, EMB), jnp.float32),
            pltpu.VMEM((B, EMB), jnp.float32),
            pltpu.VMEM((EMB,), jnp.float32),
            pltpu.SemaphoreType.DMA,
            pltpu.SemaphoreType.DMA,
            pltpu.SemaphoreType.DMA,
            pltpu.SemaphoreType.DMA,
        ],
    )
    def sc_sum(idx_hbm, emb_hbm, out_hbm,
               idx0, idx1, rows0, rows1, accv, si0, si1, sg0, sg1):
        wid = lax.axis_index("s") * NC + lax.axis_index("c")
        base = wid * chunk_rows

        def idx_copy(r, buf, sem):
            return pltpu.make_async_copy(
                idx_hbm.at[pl.ds(base + r * KJ, KJ)], buf, sem)

        def start_gathers(idxbuf, rowbuf, sem):
            for j in range(KJ):
                pltpu.make_async_copy(
                    emb_hbm.at[idxbuf.at[j]],
                    rowbuf.at[pl.ds(j * BL, BL)], sem).start()

        def wait_gathers(rowbuf, sem):
            pltpu.make_async_copy(emb_hbm.at[pl.ds(0, B)], rowbuf, sem).wait()

        def reduce_rows(rowbuf, accs):
            def body(i, accs):
                accs = list(accs)
                for j in range(KJ):
                    accs[j % 4] = accs[j % 4] + rowbuf[j * BL + i, :]
                return tuple(accs)
            return lax.fori_loop(0, BL, body, accs)

        idx_copy(0, idx0, si0).start()
        idx_copy(1, idx1, si1).start()
        idx_copy(0, idx0, si0).wait()
        start_gathers(idx0, rows0, sg0)

        zero = jnp.zeros((EMB,), jnp.float32)
        accs0 = (zero, zero, zero, zero)

        def round_pair(k, accs):
            wait_gathers(rows0, sg0)

            @pl.when(k + 1 < NK)
            def _():
                idx_copy(2 * k + 2, idx0, si0).start()

            idx_copy(2 * k + 1, idx1, si1).wait()
            start_gathers(idx1, rows1, sg1)
            accs = reduce_rows(rows0, accs)

            wait_gathers(rows1, sg1)

            @pl.when(k + 1 < NK)
            def _():
                idx_copy(2 * k + 3, idx1, si1).start()
                idx_copy(2 * k + 2, idx0, si0).wait()
                start_gathers(idx0, rows0, sg0)

            accs = reduce_rows(rows1, accs)
            return accs

        a0, a1, a2, a3 = lax.fori_loop(0, NK, round_pair, accs0)
        accv[...] = (a0 + a1) + (a2 + a3)
        pltpu.sync_copy(accv, out_hbm.at[wid])

    return sc_sum


def _tc_finish(partials, W, b2, inv_l):
    def body(p_ref, w_ref, b_ref, o_ref):
        pooled = jnp.sum(p_ref[...], axis=0, keepdims=True) * inv_l
        o_ref[...] = lax.dot_general(
            pooled, w_ref[...], (((1,), (1,)), ((), ())),
            preferred_element_type=jnp.float32) + b_ref[...]

    return pl.pallas_call(
        body,
        out_shape=jax.ShapeDtypeStruct((1, b2.shape[1]), jnp.float32),
    )(partials, W, b2)


def kernel(inputs, emb, W, b):
    L = inputs.shape[0]
    idx2d = inputs.astype(jnp.int32).reshape(L // BL, BL)
    partials = _make_sc_sum(L // BL)(idx2d, emb)
    return _tc_finish(partials, W, b.reshape(1, -1), 1.0 / L)
